# baseline (speedup 1.0000x reference)
import functools

import jax
import jax.numpy as jnp
from jax import lax
from jax.experimental import pallas as pl
from jax.experimental.pallas import tpu as pltpu
from jax.experimental.pallas import tpu_sc as plsc

N = 10000
E = 320000
D = 128
H = 128
HH = H // 2
C = 16
G = 64
SINK_ITERS = 3

NSUB = 16
NCORE = 2
NP = 10240
NODE_SLICE = NP // NSUB
EW = E // NSUB
K = 128
NCH = 160
EWP = NCH * K
NROW = NSUB * NCH
NCH2 = NROW // (NSUB * NCORE)
ROW_BLK = 512
N_ROW_BLKS = NP // ROW_BLK

_SC_PARAMS = pltpu.CompilerParams(
    needs_layout_passes=False, use_tc_tiling_on_sc=False)


def _tc_layer0_body(xlo_ref, xhi_ref, w_ref, a_ref, hlo_ref, hhi_ref, sd_ref):
    xb = jnp.concatenate([xlo_ref[...], xhi_ref[...]], axis=1)
    h = jnp.dot(xb, w_ref[...], preferred_element_type=jnp.float32)
    hlo_ref[...] = h[:, :HH]
    hhi_ref[...] = h[:, HH:]
    sd_ref[...] = jnp.dot(h, a_ref[...], preferred_element_type=jnp.float32)


def _tc_layer1_body(plo_ref, phi_ref, w_ref, a_ref, hlo_ref, hhi_ref, sd_ref):
    xb = jnp.concatenate([plo_ref[0] + plo_ref[1], phi_ref[0] + phi_ref[1]],
                         axis=1)
    xb = jnp.where(xb > 0.0, xb, jnp.exp(xb) - 1.0)
    h = jnp.dot(xb, w_ref[...], preferred_element_type=jnp.float32)
    hlo_ref[...] = h[:, :HH]
    hhi_ref[...] = h[:, HH:]
    sd_ref[...] = jnp.dot(h, a_ref[...], preferred_element_type=jnp.float32)


def _tc_layer(xlo, xhi, W, A2, partials):
    xspec = (pl.BlockSpec((2, ROW_BLK, HH), lambda i: (0, i, 0)) if partials
             else pl.BlockSpec((ROW_BLK, HH), lambda i: (i, 0)))
    return pl.pallas_call(
        _tc_layer1_body if partials else _tc_layer0_body,
        grid=(N_ROW_BLKS,),
        in_specs=[
            xspec,
            xspec,
            pl.BlockSpec((128, 128), lambda i: (0, 0)),
            pl.BlockSpec((128, 128), lambda i: (0, 0)),
        ],
        out_specs=[
            pl.BlockSpec((ROW_BLK, HH), lambda i: (i, 0)),
            pl.BlockSpec((ROW_BLK, HH), lambda i: (i, 0)),
            pl.BlockSpec((ROW_BLK, 128), lambda i: (i, 0)),
        ],
        out_shape=[
            jax.ShapeDtypeStruct((NP, HH), jnp.float32),
            jax.ShapeDtypeStruct((NP, HH), jnp.float32),
            jax.ShapeDtypeStruct((NP, 128), jnp.float32),
        ],
    )(xlo, xhi, W, A2)


def _tc_tail_body(plo_ref, phi_ref, b_ref, hw0_ref, hb0_ref, hw1_ref,
                  hb1_ref, out_ref, pooled_acc, cnt_acc):
    i = pl.program_id(0)

    @pl.when(i == 0)
    def _():
        pooled_acc[...] = jnp.zeros_like(pooled_acc)
        cnt_acc[...] = jnp.zeros_like(cnt_acc)

    hb = jnp.concatenate([plo_ref[0] + plo_ref[1], phi_ref[0] + phi_ref[1]],
                         axis=1)
    hb = jnp.where(hb > 0.0, hb, jnp.exp(hb) - 1.0)
    bidx = b_ref[0, 0, :]
    iota = lax.broadcasted_iota(jnp.int32, (G, ROW_BLK), 0)
    onehot = (bidx[None, :] == iota).astype(jnp.float32)
    pooled_acc[...] += jnp.dot(onehot, hb, preferred_element_type=jnp.float32)
    cnt_acc[...] += jnp.sum(onehot, axis=1, keepdims=True)

    @pl.when(i == N_ROW_BLKS - 1)
    def _():
        pooled = pooled_acc[...] / jnp.maximum(cnt_acc[...], 1.0)
        z = jnp.dot(pooled, hw0_ref[...], preferred_element_type=jnp.float32)
        z = jnp.maximum(z + hb0_ref[...], 0.0)
        out_ref[...] = (
            jnp.dot(z, hw1_ref[...], preferred_element_type=jnp.float32)
            + hb1_ref[...])


def _tc_tail(plo, phi, batch3d, hw0, hb0r, hw1p, hb1p):
    return pl.pallas_call(
        _tc_tail_body,
        grid=(N_ROW_BLKS,),
        in_specs=[
            pl.BlockSpec((2, ROW_BLK, HH), lambda i: (0, i, 0)),
            pl.BlockSpec((2, ROW_BLK, HH), lambda i: (0, i, 0)),
            pl.BlockSpec((1, 1, ROW_BLK), lambda i: (i, 0, 0)),
            pl.BlockSpec((128, 128), lambda i: (0, 0)),
            pl.BlockSpec((1, 128), lambda i: (0, 0)),
            pl.BlockSpec((128, 128), lambda i: (0, 0)),
            pl.BlockSpec((1, 128), lambda i: (0, 0)),
        ],
        out_specs=pl.BlockSpec((G, 128), lambda i: (0, 0)),
        out_shape=jax.ShapeDtypeStruct((G, 128), jnp.float32),
        scratch_shapes=[
            pltpu.VMEM((G, 128), jnp.float32),
            pltpu.VMEM((G, 128), jnp.float32),
        ],
    )(plo, phi, batch3d, hw0, hb0r, hw1p, hb1p)


def _sc_attn_body(s_hbm, d_hbm, src_hbm, dst_hbm, alpha_hbm,
                  src_v, dst_v, alpha_v, acc_v, zvec, acc_sh, ssem):
    wid = lax.axis_index("s")
    rbase = wid * NCH
    nbase = wid * NODE_SLICE
    zero16 = jnp.zeros((16,), jnp.float32)

    pltpu.sync_copy(src_hbm.at[pl.ds(rbase, NCH)], src_v)
    pltpu.sync_copy(dst_hbm.at[pl.ds(rbase, NCH)], dst_v)

    for j in range(8):
        zvec[pl.ds(j * 16, 16)] = zero16

    pltpu.sync_copy(s_hbm, acc_v)

    def s_body(c2, carry):
        for j in range(8):
            sl = pl.ds(j * 16, 16)
            alpha_v[c2, sl] = plsc.load_gather(acc_v, [src_v[c2, sl]])
        return carry
    lax.fori_loop(0, NCH, s_body, 0)

    pltpu.sync_copy(d_hbm, acc_v)

    def d_body(c2, carry):
        for j in range(8):
            sl = pl.ds(j * 16, 16)
            e = alpha_v[c2, sl] + plsc.load_gather(acc_v, [dst_v[c2, sl]])
            e = jnp.where(e >= 0.0, e, e * 0.2)
            alpha_v[c2, sl] = jnp.exp(e)
        return carry
    lax.fori_loop(0, NCH, d_body, 0)

    for cc in range(EW // K, NCH):
        j0 = (EW - cc * K) // 16 if cc * K < EW else 0
        for j in range(j0, 8):
            alpha_v[cc, pl.ds(j * 16, 16)] = zero16

    def seg_pass(idx_v):
        for t in range(NODE_SLICE // K):
            pltpu.sync_copy(zvec, acc_sh.at[pl.ds(nbase + t * K, K)])
        plsc.subcore_barrier()

        def scat_body(c2, carry):
            pltpu.async_copy(alpha_v.at[c2], acc_sh.at[idx_v.at[c2]], ssem,
                             add=True)
            return carry
        lax.fori_loop(0, NCH, scat_body, 0)
        pltpu.make_async_copy(src_hbm.at[pl.ds(0, NCH)], dst_v, ssem).wait()
        plsc.subcore_barrier()
        pltpu.sync_copy(acc_sh, acc_v)
        plsc.subcore_barrier()

        def div_body(c2, carry):
            for j in range(8):
                sl = pl.ds(j * 16, 16)
                r = plsc.load_gather(acc_v, [idx_v[c2, sl]])
                alpha_v[c2, sl] = alpha_v[c2, sl] / (r + 1e-9)
            return carry
        lax.fori_loop(0, NCH, div_body, 0)

    for _ in range(SINK_ITERS):
        seg_pass(dst_v)
        seg_pass(src_v)

    pltpu.sync_copy(alpha_v, alpha_hbm.at[pl.ds(rbase, NCH)])


_sc_attn = functools.partial(
    pl.kernel,
    out_type=jax.ShapeDtypeStruct((NROW, K), jnp.float32),
    mesh=plsc.VectorSubcoreMesh(
        core_axis_name="c", subcore_axis_name="s", num_cores=1),
    compiler_params=_SC_PARAMS,
    scratch_types=[
        pltpu.VMEM((NCH, K), jnp.int32),
        pltpu.VMEM((NCH, K), jnp.int32),
        pltpu.VMEM((NCH, K), jnp.float32),
        pltpu.VMEM((NP,), jnp.float32),
        pltpu.VMEM((K,), jnp.float32),
        pltpu.VMEM_SHARED((NP,), jnp.float32),
        pltpu.SemaphoreType.DMA,
    ],
)(_sc_attn_body)


def _sc_agg_body(hlo_hbm, hhi_hbm, alpha_hbm, src_hbm, dst_hbm,
                 plo_hbm, phi_hbm,
                 src_v, dst_v, alpha_v, rowbuf, rowbuf2, zvec,
                 out_sh, h_sh, gsem0, gsem1):
    cid = lax.axis_index("c")
    sid = lax.axis_index("s")
    rbase = (cid * NSUB + sid) * NCH2
    nbase = sid * NODE_SLICE
    zero16 = jnp.zeros((16,), jnp.float32)

    pltpu.sync_copy(src_hbm.at[pl.ds(rbase, NCH2)], src_v)
    pltpu.sync_copy(dst_hbm.at[pl.ds(rbase, NCH2)], dst_v)
    pltpu.sync_copy(alpha_hbm.at[pl.ds(rbase, NCH2)], alpha_v)

    for h_hbm, p_hbm in ((hlo_hbm, plo_hbm), (hhi_hbm, phi_hbm)):
        pltpu.sync_copy(h_hbm.at[pl.ds(nbase, NODE_SLICE)],
                        h_sh.at[pl.ds(nbase, NODE_SLICE)])

        def z_body(i, carry):
            for j in range(HH // 16):
                rowbuf[i, pl.ds(j * 16, 16)] = zero16
            return carry
        lax.fori_loop(0, K, z_body, 0)
        for t in range(NODE_SLICE // K):
            pltpu.sync_copy(rowbuf, out_sh.at[pl.ds(nbase + t * K, K)])
        plsc.subcore_barrier()

        bufs = ((rowbuf, gsem0), (rowbuf2, gsem1))
        for b, (buf, gsem) in enumerate(bufs):
            pltpu.async_copy(h_sh.at[src_v.at[b]], buf, gsem)

        def pipe_body(g2, carry):
            for b, (buf, gsem) in enumerate(bufs):
                c2 = 2 * g2 + b
                pltpu.make_async_copy(h_sh.at[src_v.at[c2]], buf,
                                      gsem).wait()

                def grp_body(g, carry2):
                    av = alpha_v[c2, pl.ds(g * 16, 16)]
                    for l in range(16):
                        a = av[l]
                        e2 = g * 16 + l
                        for j in range(HH // 16):
                            sl = pl.ds(j * 16, 16)
                            buf[e2, sl] = buf[e2, sl] * a
                    return carry2
                lax.fori_loop(0, K // 16, grp_body, 0)
                pltpu.sync_copy(buf, out_sh.at[dst_v.at[c2]], add=True)

                @pl.when(c2 + 2 < NCH2)
                def _():
                    pltpu.async_copy(h_sh.at[src_v.at[c2 + 2]], buf, gsem)
            return carry
        lax.fori_loop(0, NCH2 // 2, pipe_body, 0)
        plsc.subcore_barrier()

        for t in range(NODE_SLICE // K):
            pltpu.sync_copy(out_sh.at[pl.ds(nbase + t * K, K)],
                            p_hbm.at[cid, pl.ds(nbase + t * K, K)])
        plsc.subcore_barrier()


_sc_agg = functools.partial(
    pl.kernel,
    out_type=(
        jax.ShapeDtypeStruct((NCORE, NP, HH), jnp.float32),
        jax.ShapeDtypeStruct((NCORE, NP, HH), jnp.float32),
    ),
    mesh=plsc.VectorSubcoreMesh(
        core_axis_name="c", subcore_axis_name="s", num_cores=NCORE),
    compiler_params=_SC_PARAMS,
    scratch_types=[
        pltpu.VMEM((NCH2, K), jnp.int32),
        pltpu.VMEM((NCH2, K), jnp.int32),
        pltpu.VMEM((NCH2, K), jnp.float32),
        pltpu.VMEM((K, HH), jnp.float32),
        pltpu.VMEM((K, HH), jnp.float32),
        pltpu.VMEM((K,), jnp.float32),
        pltpu.VMEM_SHARED((NP, HH), jnp.float32),
        pltpu.VMEM_SHARED((NP, HH), jnp.float32),
        pltpu.SemaphoreType.DMA,
        pltpu.SemaphoreType.DMA,
    ],
)(_sc_agg_body)


def kernel(x, edge_index, batch_sample_indices,
           W0, a_src0, a_dst0, W1, a_src1, a_dst1, hw0, hb0, hw1, hb1):
    src = edge_index[0].astype(jnp.int32)
    dst = edge_index[1].astype(jnp.int32)
    src2d = jnp.pad(src.reshape(NSUB, EW), ((0, 0), (0, EWP - EW))
                    ).reshape(NROW, K)
    dst2d = jnp.pad(dst.reshape(NSUB, EW), ((0, 0), (0, EWP - EW))
                    ).reshape(NROW, K)
    xp = jnp.pad(x, ((0, NP - N), (0, 0)))
    A20 = jnp.pad(jnp.stack([a_src0, a_dst0], axis=1), ((0, 0), (0, 126)))
    A21 = jnp.pad(jnp.stack([a_src1, a_dst1], axis=1), ((0, 0), (0, 126)))

    h0lo, h0hi, sd0 = _tc_layer(xp[:, :HH], xp[:, HH:], W0, A20,
                                partials=False)
    alpha0 = _sc_attn(sd0[:, 0], sd0[:, 1], src2d, dst2d)
    p0lo, p0hi = _sc_agg(h0lo, h0hi, alpha0, src2d, dst2d)
    h1lo, h1hi, sd1 = _tc_layer(p0lo, p0hi, W1, A21, partials=True)
    alpha1 = _sc_attn(sd1[:, 0], sd1[:, 1], src2d, dst2d)
    p1lo, p1hi = _sc_agg(h1lo, h1hi, alpha1, src2d, dst2d)

    batch3d = jnp.pad(batch_sample_indices.astype(jnp.int32), (0, NP - N),
                      constant_values=G + 1).reshape(N_ROW_BLKS, 1, ROW_BLK)
    hb0r = hb0.reshape(1, 128)
    hw1p = jnp.pad(hw1, ((0, 0), (0, 128 - C)))
    hb1p = jnp.pad(hb1, (0, 128 - C)).reshape(1, 128)
    outp = _tc_tail(p1lo, p1hi, batch3d, hw0, hb0r, hw1p, hb1p)
    return outp[:, :C]


# speedup vs baseline: 1.3398x; 1.3398x over previous
import functools

import jax
import jax.numpy as jnp
from jax import lax
from jax.experimental import pallas as pl
from jax.experimental.pallas import tpu as pltpu
from jax.experimental.pallas import tpu_sc as plsc

N = 10000
E = 320000
D = 128
H = 128
HH = H // 2
C = 16
G = 64
SINK_ITERS = 3

NSUB = 16
NCORE = 2
NP = 10240
NODE_SLICE = NP // NSUB
EW = E // NSUB
K = 128
NCH = 160
EWP = NCH * K
NROW = NSUB * NCH
NCH2 = NROW // (NSUB * NCORE)
ROW_BLK = 512
N_ROW_BLKS = NP // ROW_BLK

_SC_PARAMS = pltpu.CompilerParams(
    needs_layout_passes=False, use_tc_tiling_on_sc=False)


def _tc_layer0_body(xlo_ref, xhi_ref, w_ref, a_ref, hlo_ref, hhi_ref, sd_ref):
    xb = jnp.concatenate([xlo_ref[...], xhi_ref[...]], axis=1)
    h = jnp.dot(xb, w_ref[...], preferred_element_type=jnp.float32)
    hlo_ref[...] = h[:, :HH]
    hhi_ref[...] = h[:, HH:]
    sd_ref[...] = jnp.dot(h, a_ref[...], preferred_element_type=jnp.float32)


def _tc_layer1_body(plo_ref, phi_ref, w_ref, a_ref, hlo_ref, hhi_ref, sd_ref):
    xb = jnp.concatenate([plo_ref[0] + plo_ref[1], phi_ref[0] + phi_ref[1]],
                         axis=1)
    xb = jnp.where(xb > 0.0, xb, jnp.exp(xb) - 1.0)
    h = jnp.dot(xb, w_ref[...], preferred_element_type=jnp.float32)
    hlo_ref[...] = h[:, :HH]
    hhi_ref[...] = h[:, HH:]
    sd_ref[...] = jnp.dot(h, a_ref[...], preferred_element_type=jnp.float32)


def _tc_layer(xlo, xhi, W, A2, partials):
    xspec = (pl.BlockSpec((2, ROW_BLK, HH), lambda i: (0, i, 0)) if partials
             else pl.BlockSpec((ROW_BLK, HH), lambda i: (i, 0)))
    return pl.pallas_call(
        _tc_layer1_body if partials else _tc_layer0_body,
        grid=(N_ROW_BLKS,),
        in_specs=[
            xspec,
            xspec,
            pl.BlockSpec((128, 128), lambda i: (0, 0)),
            pl.BlockSpec((128, 128), lambda i: (0, 0)),
        ],
        out_specs=[
            pl.BlockSpec((ROW_BLK, HH), lambda i: (i, 0)),
            pl.BlockSpec((ROW_BLK, HH), lambda i: (i, 0)),
            pl.BlockSpec((ROW_BLK, 128), lambda i: (i, 0)),
        ],
        out_shape=[
            jax.ShapeDtypeStruct((NP, HH), jnp.float32),
            jax.ShapeDtypeStruct((NP, HH), jnp.float32),
            jax.ShapeDtypeStruct((NP, 128), jnp.float32),
        ],
    )(xlo, xhi, W, A2)


def _tc_tail_body(plo_ref, phi_ref, b_ref, hw0_ref, hb0_ref, hw1_ref,
                  hb1_ref, out_ref, pooled_acc, cnt_acc):
    i = pl.program_id(0)

    @pl.when(i == 0)
    def _():
        pooled_acc[...] = jnp.zeros_like(pooled_acc)
        cnt_acc[...] = jnp.zeros_like(cnt_acc)

    hb = jnp.concatenate([plo_ref[0] + plo_ref[1], phi_ref[0] + phi_ref[1]],
                         axis=1)
    hb = jnp.where(hb > 0.0, hb, jnp.exp(hb) - 1.0)
    bidx = b_ref[0, 0, :]
    iota = lax.broadcasted_iota(jnp.int32, (G, ROW_BLK), 0)
    onehot = (bidx[None, :] == iota).astype(jnp.float32)
    pooled_acc[...] += jnp.dot(onehot, hb, preferred_element_type=jnp.float32)
    cnt_acc[...] += jnp.sum(onehot, axis=1, keepdims=True)

    @pl.when(i == N_ROW_BLKS - 1)
    def _():
        pooled = pooled_acc[...] / jnp.maximum(cnt_acc[...], 1.0)
        z = jnp.dot(pooled, hw0_ref[...], preferred_element_type=jnp.float32)
        z = jnp.maximum(z + hb0_ref[...], 0.0)
        out_ref[...] = (
            jnp.dot(z, hw1_ref[...], preferred_element_type=jnp.float32)
            + hb1_ref[...])


def _tc_tail(plo, phi, batch3d, hw0, hb0r, hw1p, hb1p):
    return pl.pallas_call(
        _tc_tail_body,
        grid=(N_ROW_BLKS,),
        in_specs=[
            pl.BlockSpec((2, ROW_BLK, HH), lambda i: (0, i, 0)),
            pl.BlockSpec((2, ROW_BLK, HH), lambda i: (0, i, 0)),
            pl.BlockSpec((1, 1, ROW_BLK), lambda i: (i, 0, 0)),
            pl.BlockSpec((128, 128), lambda i: (0, 0)),
            pl.BlockSpec((1, 128), lambda i: (0, 0)),
            pl.BlockSpec((128, 128), lambda i: (0, 0)),
            pl.BlockSpec((1, 128), lambda i: (0, 0)),
        ],
        out_specs=pl.BlockSpec((G, 128), lambda i: (0, 0)),
        out_shape=jax.ShapeDtypeStruct((G, 128), jnp.float32),
        scratch_shapes=[
            pltpu.VMEM((G, 128), jnp.float32),
            pltpu.VMEM((G, 128), jnp.float32),
        ],
    )(plo, phi, batch3d, hw0, hb0r, hw1p, hb1p)


def _sc_attn_body(s_hbm, d_hbm, src_hbm, dst_hbm, alpha_hbm,
                  src_v, dst_v, alpha_v, acc_v, zvec, acc_sh, ssem):
    wid = lax.axis_index("s")
    rbase = wid * NCH
    nbase = wid * NODE_SLICE
    zero16 = jnp.zeros((16,), jnp.float32)

    pltpu.sync_copy(src_hbm.at[pl.ds(rbase, NCH)], src_v)
    pltpu.sync_copy(dst_hbm.at[pl.ds(rbase, NCH)], dst_v)

    for j in range(8):
        zvec[pl.ds(j * 16, 16)] = zero16

    pltpu.sync_copy(s_hbm, acc_v)

    def s_body(c2, carry):
        for j in range(8):
            sl = pl.ds(j * 16, 16)
            alpha_v[c2, sl] = plsc.load_gather(acc_v, [src_v[c2, sl]])
        return carry
    lax.fori_loop(0, NCH, s_body, 0)

    pltpu.sync_copy(d_hbm, acc_v)

    def d_body(c2, carry):
        for j in range(8):
            sl = pl.ds(j * 16, 16)
            e = alpha_v[c2, sl] + plsc.load_gather(acc_v, [dst_v[c2, sl]])
            e = jnp.where(e >= 0.0, e, e * 0.2)
            alpha_v[c2, sl] = jnp.exp(e)
        return carry
    lax.fori_loop(0, NCH, d_body, 0)

    for cc in range(EW // K, NCH):
        j0 = (EW - cc * K) // 16 if cc * K < EW else 0
        for j in range(j0, 8):
            alpha_v[cc, pl.ds(j * 16, 16)] = zero16

    def seg_pass(idx_v):
        for t in range(NODE_SLICE // K):
            pltpu.sync_copy(zvec, acc_sh.at[pl.ds(nbase + t * K, K)])
        plsc.subcore_barrier()

        def scat_body(c2, carry):
            pltpu.async_copy(alpha_v.at[c2], acc_sh.at[idx_v.at[c2]], ssem,
                             add=True)
            return carry
        lax.fori_loop(0, NCH, scat_body, 0)
        pltpu.make_async_copy(src_hbm.at[pl.ds(0, NCH)], dst_v, ssem).wait()
        plsc.subcore_barrier()
        pltpu.sync_copy(acc_sh, acc_v)
        plsc.subcore_barrier()

        def div_body(c2, carry):
            for j in range(8):
                sl = pl.ds(j * 16, 16)
                r = plsc.load_gather(acc_v, [idx_v[c2, sl]])
                alpha_v[c2, sl] = alpha_v[c2, sl] / (r + 1e-9)
            return carry
        lax.fori_loop(0, NCH, div_body, 0)

    for _ in range(SINK_ITERS):
        seg_pass(dst_v)
        seg_pass(src_v)

    pltpu.sync_copy(alpha_v, alpha_hbm.at[pl.ds(rbase, NCH)])


_sc_attn = functools.partial(
    pl.kernel,
    out_type=jax.ShapeDtypeStruct((NROW, K), jnp.float32),
    mesh=plsc.VectorSubcoreMesh(
        core_axis_name="c", subcore_axis_name="s", num_cores=1),
    compiler_params=_SC_PARAMS,
    scratch_types=[
        pltpu.VMEM((NCH, K), jnp.int32),
        pltpu.VMEM((NCH, K), jnp.int32),
        pltpu.VMEM((NCH, K), jnp.float32),
        pltpu.VMEM((NP,), jnp.float32),
        pltpu.VMEM((K,), jnp.float32),
        pltpu.VMEM_SHARED((NP,), jnp.float32),
        pltpu.SemaphoreType.DMA,
    ],
)(_sc_attn_body)


def _sc_agg_body(hlo_hbm, hhi_hbm, alpha_hbm, src_hbm, dst_hbm,
                 plo_hbm, phi_hbm,
                 src_v, dst_v, alpha_v, rowbuf, rowbuf2, zvec,
                 out_sh, h_sh, gsem0, gsem1):
    cid = lax.axis_index("c")
    sid = lax.axis_index("s")
    rbase = (cid * NSUB + sid) * NCH2
    nbase = sid * NODE_SLICE
    zero16 = jnp.zeros((16,), jnp.float32)

    pltpu.sync_copy(src_hbm.at[pl.ds(rbase, NCH2)], src_v)
    pltpu.sync_copy(dst_hbm.at[pl.ds(rbase, NCH2)], dst_v)
    pltpu.sync_copy(alpha_hbm.at[pl.ds(rbase, NCH2)], alpha_v)

    for h_hbm, p_hbm in ((hlo_hbm, plo_hbm), (hhi_hbm, phi_hbm)):
        pltpu.sync_copy(h_hbm.at[pl.ds(nbase, NODE_SLICE)],
                        h_sh.at[pl.ds(nbase, NODE_SLICE)])

        def z_body(i, carry):
            for j in range(HH // 16):
                rowbuf[i, pl.ds(j * 16, 16)] = zero16
            return carry
        lax.fori_loop(0, K, z_body, 0)
        for t in range(NODE_SLICE // K):
            pltpu.sync_copy(rowbuf, out_sh.at[pl.ds(nbase + t * K, K)])
        plsc.subcore_barrier()

        bufs = ((rowbuf, gsem0), (rowbuf2, gsem1))
        for b, (buf, gsem) in enumerate(bufs):
            pltpu.async_copy(h_sh.at[src_v.at[b]], buf, gsem)

        def pipe_body(g2, carry):
            for b, (buf, gsem) in enumerate(bufs):
                c2 = 2 * g2 + b
                pltpu.make_async_copy(h_sh.at[src_v.at[c2]], buf,
                                      gsem).wait()

                c2v = jnp.full((16,), c2, jnp.int32)

                def grp_body(g, carry2):
                    gbase = jnp.full((16,), g * 16, jnp.int32)
                    for l in range(16):
                        e2 = g * 16 + l
                        a16 = plsc.load_gather(alpha_v, [c2v, gbase + l])
                        for j in range(HH // 16):
                            sl = pl.ds(j * 16, 16)
                            buf[e2, sl] = buf[e2, sl] * a16
                    return carry2
                lax.fori_loop(0, K // 16, grp_body, 0)
                pltpu.sync_copy(buf, out_sh.at[dst_v.at[c2]], add=True)

                @pl.when(c2 + 2 < NCH2)
                def _():
                    pltpu.async_copy(h_sh.at[src_v.at[c2 + 2]], buf, gsem)
            return carry
        lax.fori_loop(0, NCH2 // 2, pipe_body, 0)
        plsc.subcore_barrier()

        for t in range(NODE_SLICE // K):
            pltpu.sync_copy(out_sh.at[pl.ds(nbase + t * K, K)],
                            p_hbm.at[cid, pl.ds(nbase + t * K, K)])
        plsc.subcore_barrier()


_sc_agg = functools.partial(
    pl.kernel,
    out_type=(
        jax.ShapeDtypeStruct((NCORE, NP, HH), jnp.float32),
        jax.ShapeDtypeStruct((NCORE, NP, HH), jnp.float32),
    ),
    mesh=plsc.VectorSubcoreMesh(
        core_axis_name="c", subcore_axis_name="s", num_cores=NCORE),
    compiler_params=_SC_PARAMS,
    scratch_types=[
        pltpu.VMEM((NCH2, K), jnp.int32),
        pltpu.VMEM((NCH2, K), jnp.int32),
        pltpu.VMEM((NCH2, K), jnp.float32),
        pltpu.VMEM((K, HH), jnp.float32),
        pltpu.VMEM((K, HH), jnp.float32),
        pltpu.VMEM((K,), jnp.float32),
        pltpu.VMEM_SHARED((NP, HH), jnp.float32),
        pltpu.VMEM_SHARED((NP, HH), jnp.float32),
        pltpu.SemaphoreType.DMA,
        pltpu.SemaphoreType.DMA,
    ],
)(_sc_agg_body)


def kernel(x, edge_index, batch_sample_indices,
           W0, a_src0, a_dst0, W1, a_src1, a_dst1, hw0, hb0, hw1, hb1):
    src = edge_index[0].astype(jnp.int32)
    dst = edge_index[1].astype(jnp.int32)
    src2d = jnp.pad(src.reshape(NSUB, EW), ((0, 0), (0, EWP - EW))
                    ).reshape(NROW, K)
    dst2d = jnp.pad(dst.reshape(NSUB, EW), ((0, 0), (0, EWP - EW))
                    ).reshape(NROW, K)
    xp = jnp.pad(x, ((0, NP - N), (0, 0)))
    A20 = jnp.pad(jnp.stack([a_src0, a_dst0], axis=1), ((0, 0), (0, 126)))
    A21 = jnp.pad(jnp.stack([a_src1, a_dst1], axis=1), ((0, 0), (0, 126)))

    h0lo, h0hi, sd0 = _tc_layer(xp[:, :HH], xp[:, HH:], W0, A20,
                                partials=False)
    alpha0 = _sc_attn(sd0[:, 0], sd0[:, 1], src2d, dst2d)
    p0lo, p0hi = _sc_agg(h0lo, h0hi, alpha0, src2d, dst2d)
    h1lo, h1hi, sd1 = _tc_layer(p0lo, p0hi, W1, A21, partials=True)
    alpha1 = _sc_attn(sd1[:, 0], sd1[:, 1], src2d, dst2d)
    p1lo, p1hi = _sc_agg(h1lo, h1hi, alpha1, src2d, dst2d)

    batch3d = jnp.pad(batch_sample_indices.astype(jnp.int32), (0, NP - N),
                      constant_values=G + 1).reshape(N_ROW_BLKS, 1, ROW_BLK)
    hb0r = hb0.reshape(1, 128)
    hw1p = jnp.pad(hw1, ((0, 0), (0, 128 - C)))
    hb1p = jnp.pad(hb1, (0, 128 - C)).reshape(1, 128)
    outp = _tc_tail(p1lo, p1hi, batch3d, hw0, hb0r, hw1p, hb1p)
    return outp[:, :C]


# speedup vs baseline: 1.3755x; 1.0266x over previous
import functools

import jax
import jax.numpy as jnp
from jax import lax
from jax.experimental import pallas as pl
from jax.experimental.pallas import tpu as pltpu
from jax.experimental.pallas import tpu_sc as plsc

N = 10000
E = 320000
D = 128
H = 128
HH = H // 2
C = 16
G = 64
SINK_ITERS = 3

NSUB = 16
NCORE = 2
NP = 10240
NODE_SLICE = NP // NSUB
EW = E // NSUB
K = 128
NCH = 160
EWP = NCH * K
NROW = NSUB * NCH
NCH2 = NROW // (NSUB * NCORE)
ROW_BLK = 512
N_ROW_BLKS = NP // ROW_BLK

_SC_PARAMS = pltpu.CompilerParams(
    needs_layout_passes=False, use_tc_tiling_on_sc=False)


def _tc_layer0_body(xlo_ref, xhi_ref, w_ref, a_ref, hlo_ref, hhi_ref, sd_ref):
    xb = jnp.concatenate([xlo_ref[...], xhi_ref[...]], axis=1)
    h = jnp.dot(xb, w_ref[...], preferred_element_type=jnp.float32)
    hlo_ref[...] = h[:, :HH]
    hhi_ref[...] = h[:, HH:]
    sd_ref[...] = jnp.dot(h, a_ref[...], preferred_element_type=jnp.float32)


def _tc_layer1_body(plo_ref, phi_ref, w_ref, a_ref, hlo_ref, hhi_ref, sd_ref):
    xb = jnp.concatenate([plo_ref[0] + plo_ref[1], phi_ref[0] + phi_ref[1]],
                         axis=1)
    xb = jnp.where(xb > 0.0, xb, jnp.exp(xb) - 1.0)
    h = jnp.dot(xb, w_ref[...], preferred_element_type=jnp.float32)
    hlo_ref[...] = h[:, :HH]
    hhi_ref[...] = h[:, HH:]
    sd_ref[...] = jnp.dot(h, a_ref[...], preferred_element_type=jnp.float32)


def _tc_layer(xlo, xhi, W, A2, partials):
    xspec = (pl.BlockSpec((2, ROW_BLK, HH), lambda i: (0, i, 0)) if partials
             else pl.BlockSpec((ROW_BLK, HH), lambda i: (i, 0)))
    return pl.pallas_call(
        _tc_layer1_body if partials else _tc_layer0_body,
        grid=(N_ROW_BLKS,),
        in_specs=[
            xspec,
            xspec,
            pl.BlockSpec((128, 128), lambda i: (0, 0)),
            pl.BlockSpec((128, 128), lambda i: (0, 0)),
        ],
        out_specs=[
            pl.BlockSpec((ROW_BLK, HH), lambda i: (i, 0)),
            pl.BlockSpec((ROW_BLK, HH), lambda i: (i, 0)),
            pl.BlockSpec((ROW_BLK, 128), lambda i: (i, 0)),
        ],
        out_shape=[
            jax.ShapeDtypeStruct((NP, HH), jnp.float32),
            jax.ShapeDtypeStruct((NP, HH), jnp.float32),
            jax.ShapeDtypeStruct((NP, 128), jnp.float32),
        ],
    )(xlo, xhi, W, A2)


def _tc_tail_body(plo_ref, phi_ref, b_ref, hw0_ref, hb0_ref, hw1_ref,
                  hb1_ref, out_ref, pooled_acc, cnt_acc):
    i = pl.program_id(0)

    @pl.when(i == 0)
    def _():
        pooled_acc[...] = jnp.zeros_like(pooled_acc)
        cnt_acc[...] = jnp.zeros_like(cnt_acc)

    hb = jnp.concatenate([plo_ref[0] + plo_ref[1], phi_ref[0] + phi_ref[1]],
                         axis=1)
    hb = jnp.where(hb > 0.0, hb, jnp.exp(hb) - 1.0)
    bidx = b_ref[0, 0, :]
    iota = lax.broadcasted_iota(jnp.int32, (G, ROW_BLK), 0)
    onehot = (bidx[None, :] == iota).astype(jnp.float32)
    pooled_acc[...] += jnp.dot(onehot, hb, preferred_element_type=jnp.float32)
    cnt_acc[...] += jnp.sum(onehot, axis=1, keepdims=True)

    @pl.when(i == N_ROW_BLKS - 1)
    def _():
        pooled = pooled_acc[...] / jnp.maximum(cnt_acc[...], 1.0)
        z = jnp.dot(pooled, hw0_ref[...], preferred_element_type=jnp.float32)
        z = jnp.maximum(z + hb0_ref[...], 0.0)
        out_ref[...] = (
            jnp.dot(z, hw1_ref[...], preferred_element_type=jnp.float32)
            + hb1_ref[...])


def _tc_tail(plo, phi, batch3d, hw0, hb0r, hw1p, hb1p):
    return pl.pallas_call(
        _tc_tail_body,
        grid=(N_ROW_BLKS,),
        in_specs=[
            pl.BlockSpec((2, ROW_BLK, HH), lambda i: (0, i, 0)),
            pl.BlockSpec((2, ROW_BLK, HH), lambda i: (0, i, 0)),
            pl.BlockSpec((1, 1, ROW_BLK), lambda i: (i, 0, 0)),
            pl.BlockSpec((128, 128), lambda i: (0, 0)),
            pl.BlockSpec((1, 128), lambda i: (0, 0)),
            pl.BlockSpec((128, 128), lambda i: (0, 0)),
            pl.BlockSpec((1, 128), lambda i: (0, 0)),
        ],
        out_specs=pl.BlockSpec((G, 128), lambda i: (0, 0)),
        out_shape=jax.ShapeDtypeStruct((G, 128), jnp.float32),
        scratch_shapes=[
            pltpu.VMEM((G, 128), jnp.float32),
            pltpu.VMEM((G, 128), jnp.float32),
        ],
    )(plo, phi, batch3d, hw0, hb0r, hw1p, hb1p)


def _sc_attn_body(s_hbm, d_hbm, src_hbm, dst_hbm, alpha_hbm,
                  src_v, dst_v, alpha_v, snode_v, acc_v, zvec,
                  acc_shA, acc_shB, ssem):
    wid = lax.axis_index("s")
    rbase = wid * NCH
    nbase = wid * NODE_SLICE
    zero16 = jnp.zeros((16,), jnp.float32)

    pltpu.sync_copy(src_hbm.at[pl.ds(rbase, NCH)], src_v)
    pltpu.sync_copy(dst_hbm.at[pl.ds(rbase, NCH)], dst_v)
    pltpu.sync_copy(s_hbm, snode_v)
    pltpu.sync_copy(d_hbm, acc_v)

    for j in range(NODE_SLICE // 16):
        zvec[pl.ds(j * 16, 16)] = zero16

    def a_body(c2, carry):
        for j in range(8):
            sl = pl.ds(j * 16, 16)
            e = (plsc.load_gather(snode_v, [src_v[c2, sl]])
                 + plsc.load_gather(acc_v, [dst_v[c2, sl]]))
            e = jnp.where(e >= 0.0, e, e * 0.2)
            alpha_v[c2, sl] = jnp.exp(e)
        return carry
    lax.fori_loop(0, NCH, a_body, 0)

    for cc in range(EW // K, NCH):
        j0 = (EW - cc * K) // 16 if cc * K < EW else 0
        for j in range(j0, 8):
            alpha_v[cc, pl.ds(j * 16, 16)] = zero16

    def seg_pass(idx_v, acc_sh):
        pltpu.sync_copy(zvec, acc_sh.at[pl.ds(nbase, NODE_SLICE)])
        plsc.subcore_barrier()

        def scat_body(c2, carry):
            pltpu.async_copy(alpha_v.at[c2], acc_sh.at[idx_v.at[c2]], ssem,
                             add=True)
            return carry
        lax.fori_loop(0, NCH, scat_body, 0)
        pltpu.make_async_copy(src_hbm.at[pl.ds(0, NCH)], dst_v, ssem).wait()
        plsc.subcore_barrier()
        pltpu.sync_copy(acc_sh, acc_v)

        def rec_body(c2, carry):
            sl = pl.ds(c2 * 16, 16)
            acc_v[sl] = 1.0 / (acc_v[sl] + 1e-9)
            return carry
        lax.fori_loop(0, NP // 16, rec_body, 0)

        def div_body(c2, carry):
            for j in range(8):
                sl = pl.ds(j * 16, 16)
                r = plsc.load_gather(acc_v, [idx_v[c2, sl]])
                alpha_v[c2, sl] = alpha_v[c2, sl] * r
            return carry
        lax.fori_loop(0, NCH, div_body, 0)

    for _ in range(SINK_ITERS):
        seg_pass(dst_v, acc_shA)
        seg_pass(src_v, acc_shB)

    pltpu.sync_copy(alpha_v, alpha_hbm.at[pl.ds(rbase, NCH)])


_sc_attn = functools.partial(
    pl.kernel,
    out_type=jax.ShapeDtypeStruct((NROW, K), jnp.float32),
    mesh=plsc.VectorSubcoreMesh(
        core_axis_name="c", subcore_axis_name="s", num_cores=1),
    compiler_params=_SC_PARAMS,
    scratch_types=[
        pltpu.VMEM((NCH, K), jnp.int32),
        pltpu.VMEM((NCH, K), jnp.int32),
        pltpu.VMEM((NCH, K), jnp.float32),
        pltpu.VMEM((NP,), jnp.float32),
        pltpu.VMEM((NP,), jnp.float32),
        pltpu.VMEM((NODE_SLICE,), jnp.float32),
        pltpu.VMEM_SHARED((NP,), jnp.float32),
        pltpu.VMEM_SHARED((NP,), jnp.float32),
        pltpu.SemaphoreType.DMA,
    ],
)(_sc_attn_body)


def _sc_agg_body(hlo_hbm, hhi_hbm, alpha_hbm, src_hbm, dst_hbm,
                 plo_hbm, phi_hbm,
                 src_v, dst_v, alpha_v, rowbuf, rowbuf2, zvec,
                 out_sh, h_sh, gsem0, gsem1):
    cid = lax.axis_index("c")
    sid = lax.axis_index("s")
    rbase = (cid * NSUB + sid) * NCH2
    nbase = sid * NODE_SLICE
    zero16 = jnp.zeros((16,), jnp.float32)

    pltpu.sync_copy(src_hbm.at[pl.ds(rbase, NCH2)], src_v)
    pltpu.sync_copy(dst_hbm.at[pl.ds(rbase, NCH2)], dst_v)
    pltpu.sync_copy(alpha_hbm.at[pl.ds(rbase, NCH2)], alpha_v)

    for h_hbm, p_hbm in ((hlo_hbm, plo_hbm), (hhi_hbm, phi_hbm)):
        pltpu.sync_copy(h_hbm.at[pl.ds(nbase, NODE_SLICE)],
                        h_sh.at[pl.ds(nbase, NODE_SLICE)])

        def z_body(i, carry):
            for j in range(HH // 16):
                rowbuf[i, pl.ds(j * 16, 16)] = zero16
            return carry
        lax.fori_loop(0, K, z_body, 0)
        for t in range(NODE_SLICE // K):
            pltpu.sync_copy(rowbuf, out_sh.at[pl.ds(nbase + t * K, K)])
        plsc.subcore_barrier()

        bufs = ((rowbuf, gsem0), (rowbuf2, gsem1))
        for b, (buf, gsem) in enumerate(bufs):
            pltpu.async_copy(h_sh.at[src_v.at[b]], buf, gsem)

        def pipe_body(g2, carry):
            for b, (buf, gsem) in enumerate(bufs):
                c2 = 2 * g2 + b
                pltpu.make_async_copy(h_sh.at[src_v.at[c2]], buf,
                                      gsem).wait()

                c2v = jnp.full((16,), c2, jnp.int32)

                def grp_body(g, carry2):
                    gbase = jnp.full((16,), g * 16, jnp.int32)
                    for l in range(16):
                        e2 = g * 16 + l
                        a16 = plsc.load_gather(alpha_v, [c2v, gbase + l])
                        for j in range(HH // 16):
                            sl = pl.ds(j * 16, 16)
                            buf[e2, sl] = buf[e2, sl] * a16
                    return carry2
                lax.fori_loop(0, K // 16, grp_body, 0)
                pltpu.sync_copy(buf, out_sh.at[dst_v.at[c2]], add=True)

                @pl.when(c2 + 2 < NCH2)
                def _():
                    pltpu.async_copy(h_sh.at[src_v.at[c2 + 2]], buf, gsem)
            return carry
        lax.fori_loop(0, NCH2 // 2, pipe_body, 0)
        plsc.subcore_barrier()

        for t in range(NODE_SLICE // K):
            pltpu.sync_copy(out_sh.at[pl.ds(nbase + t * K, K)],
                            p_hbm.at[cid, pl.ds(nbase + t * K, K)])
        plsc.subcore_barrier()


_sc_agg = functools.partial(
    pl.kernel,
    out_type=(
        jax.ShapeDtypeStruct((NCORE, NP, HH), jnp.float32),
        jax.ShapeDtypeStruct((NCORE, NP, HH), jnp.float32),
    ),
    mesh=plsc.VectorSubcoreMesh(
        core_axis_name="c", subcore_axis_name="s", num_cores=NCORE),
    compiler_params=_SC_PARAMS,
    scratch_types=[
        pltpu.VMEM((NCH2, K), jnp.int32),
        pltpu.VMEM((NCH2, K), jnp.int32),
        pltpu.VMEM((NCH2, K), jnp.float32),
        pltpu.VMEM((K, HH), jnp.float32),
        pltpu.VMEM((K, HH), jnp.float32),
        pltpu.VMEM((K,), jnp.float32),
        pltpu.VMEM_SHARED((NP, HH), jnp.float32),
        pltpu.VMEM_SHARED((NP, HH), jnp.float32),
        pltpu.SemaphoreType.DMA,
        pltpu.SemaphoreType.DMA,
    ],
)(_sc_agg_body)


def kernel(x, edge_index, batch_sample_indices,
           W0, a_src0, a_dst0, W1, a_src1, a_dst1, hw0, hb0, hw1, hb1):
    src = edge_index[0].astype(jnp.int32)
    dst = edge_index[1].astype(jnp.int32)
    src2d = jnp.pad(src.reshape(NSUB, EW), ((0, 0), (0, EWP - EW))
                    ).reshape(NROW, K)
    dst2d = jnp.pad(dst.reshape(NSUB, EW), ((0, 0), (0, EWP - EW))
                    ).reshape(NROW, K)
    xp = jnp.pad(x, ((0, NP - N), (0, 0)))
    A20 = jnp.pad(jnp.stack([a_src0, a_dst0], axis=1), ((0, 0), (0, 126)))
    A21 = jnp.pad(jnp.stack([a_src1, a_dst1], axis=1), ((0, 0), (0, 126)))

    h0lo, h0hi, sd0 = _tc_layer(xp[:, :HH], xp[:, HH:], W0, A20,
                                partials=False)
    alpha0 = _sc_attn(sd0[:, 0], sd0[:, 1], src2d, dst2d)
    p0lo, p0hi = _sc_agg(h0lo, h0hi, alpha0, src2d, dst2d)
    h1lo, h1hi, sd1 = _tc_layer(p0lo, p0hi, W1, A21, partials=True)
    alpha1 = _sc_attn(sd1[:, 0], sd1[:, 1], src2d, dst2d)
    p1lo, p1hi = _sc_agg(h1lo, h1hi, alpha1, src2d, dst2d)

    batch3d = jnp.pad(batch_sample_indices.astype(jnp.int32), (0, NP - N),
                      constant_values=G + 1).reshape(N_ROW_BLKS, 1, ROW_BLK)
    hb0r = hb0.reshape(1, 128)
    hw1p = jnp.pad(hw1, ((0, 0), (0, 128 - C)))
    hb1p = jnp.pad(hb1, (0, 128 - C)).reshape(1, 128)
    outp = _tc_tail(p1lo, p1hi, batch3d, hw0, hb0r, hw1p, hb1p)
    return outp[:, :C]


# speedup vs baseline: 1.3830x; 1.0055x over previous
import functools

import jax
import jax.numpy as jnp
from jax import lax
from jax.experimental import pallas as pl
from jax.experimental.pallas import tpu as pltpu
from jax.experimental.pallas import tpu_sc as plsc

N = 10000
E = 320000
D = 128
H = 128
HH = H // 2
C = 16
G = 64
SINK_ITERS = 3

NSUB = 16
NCORE = 2
NP = 10240
NODE_SLICE = NP // NSUB
EW = E // NSUB
K = 128
NCH = 160
EWP = NCH * K
NROW = NSUB * NCH
NCH2 = NROW // (NSUB * NCORE)
ROW_BLK = 512
N_ROW_BLKS = NP // ROW_BLK

_SC_PARAMS = pltpu.CompilerParams(
    needs_layout_passes=False, use_tc_tiling_on_sc=False)


def _tc_layer0_body(xlo_ref, xhi_ref, w_ref, a_ref, hlo_ref, hhi_ref, sd_ref):
    xb = jnp.concatenate([xlo_ref[...], xhi_ref[...]], axis=1)
    h = jnp.dot(xb, w_ref[...], preferred_element_type=jnp.float32)
    hlo_ref[...] = h[:, :HH]
    hhi_ref[...] = h[:, HH:]
    sd_ref[...] = jnp.dot(h, a_ref[...], preferred_element_type=jnp.float32)


def _tc_layer1_body(plo_ref, phi_ref, w_ref, a_ref, hlo_ref, hhi_ref, sd_ref):
    xb = jnp.concatenate([plo_ref[0] + plo_ref[1], phi_ref[0] + phi_ref[1]],
                         axis=1)
    xb = jnp.where(xb > 0.0, xb, jnp.exp(xb) - 1.0)
    h = jnp.dot(xb, w_ref[...], preferred_element_type=jnp.float32)
    hlo_ref[...] = h[:, :HH]
    hhi_ref[...] = h[:, HH:]
    sd_ref[...] = jnp.dot(h, a_ref[...], preferred_element_type=jnp.float32)


def _tc_layer(xlo, xhi, W, A2, partials):
    xspec = (pl.BlockSpec((2, ROW_BLK, HH), lambda i: (0, i, 0)) if partials
             else pl.BlockSpec((ROW_BLK, HH), lambda i: (i, 0)))
    return pl.pallas_call(
        _tc_layer1_body if partials else _tc_layer0_body,
        grid=(N_ROW_BLKS,),
        in_specs=[
            xspec,
            xspec,
            pl.BlockSpec((128, 128), lambda i: (0, 0)),
            pl.BlockSpec((128, 128), lambda i: (0, 0)),
        ],
        out_specs=[
            pl.BlockSpec((ROW_BLK, HH), lambda i: (i, 0)),
            pl.BlockSpec((ROW_BLK, HH), lambda i: (i, 0)),
            pl.BlockSpec((ROW_BLK, 128), lambda i: (i, 0)),
        ],
        out_shape=[
            jax.ShapeDtypeStruct((NP, HH), jnp.float32),
            jax.ShapeDtypeStruct((NP, HH), jnp.float32),
            jax.ShapeDtypeStruct((NP, 128), jnp.float32),
        ],
    )(xlo, xhi, W, A2)


def _tc_tail_body(plo_ref, phi_ref, b_ref, hw0_ref, hb0_ref, hw1_ref,
                  hb1_ref, out_ref, pooled_acc, cnt_acc):
    i = pl.program_id(0)

    @pl.when(i == 0)
    def _():
        pooled_acc[...] = jnp.zeros_like(pooled_acc)
        cnt_acc[...] = jnp.zeros_like(cnt_acc)

    hb = jnp.concatenate([plo_ref[0] + plo_ref[1], phi_ref[0] + phi_ref[1]],
                         axis=1)
    hb = jnp.where(hb > 0.0, hb, jnp.exp(hb) - 1.0)
    bidx = b_ref[0, 0, :]
    iota = lax.broadcasted_iota(jnp.int32, (G, ROW_BLK), 0)
    onehot = (bidx[None, :] == iota).astype(jnp.float32)
    pooled_acc[...] += jnp.dot(onehot, hb, preferred_element_type=jnp.float32)
    cnt_acc[...] += jnp.sum(onehot, axis=1, keepdims=True)

    @pl.when(i == N_ROW_BLKS - 1)
    def _():
        pooled = pooled_acc[...] / jnp.maximum(cnt_acc[...], 1.0)
        z = jnp.dot(pooled, hw0_ref[...], preferred_element_type=jnp.float32)
        z = jnp.maximum(z + hb0_ref[...], 0.0)
        out_ref[...] = (
            jnp.dot(z, hw1_ref[...], preferred_element_type=jnp.float32)
            + hb1_ref[...])


def _tc_tail(plo, phi, batch3d, hw0, hb0r, hw1p, hb1p):
    return pl.pallas_call(
        _tc_tail_body,
        grid=(N_ROW_BLKS,),
        in_specs=[
            pl.BlockSpec((2, ROW_BLK, HH), lambda i: (0, i, 0)),
            pl.BlockSpec((2, ROW_BLK, HH), lambda i: (0, i, 0)),
            pl.BlockSpec((1, 1, ROW_BLK), lambda i: (i, 0, 0)),
            pl.BlockSpec((128, 128), lambda i: (0, 0)),
            pl.BlockSpec((1, 128), lambda i: (0, 0)),
            pl.BlockSpec((128, 128), lambda i: (0, 0)),
            pl.BlockSpec((1, 128), lambda i: (0, 0)),
        ],
        out_specs=pl.BlockSpec((G, 128), lambda i: (0, 0)),
        out_shape=jax.ShapeDtypeStruct((G, 128), jnp.float32),
        scratch_shapes=[
            pltpu.VMEM((G, 128), jnp.float32),
            pltpu.VMEM((G, 128), jnp.float32),
        ],
    )(plo, phi, batch3d, hw0, hb0r, hw1p, hb1p)


def _sc_attn_body(s_hbm, d_hbm, src_hbm, dst_hbm, alpha_hbm,
                  src_v, dst_v, alpha_v, snode_v, acc_v, lacc_v, idx_id,
                  zvec, acc_shA, acc_shB, ssem):
    wid = lax.axis_index("s")
    rbase = wid * NCH
    nbase = wid * NODE_SLICE
    zero16 = jnp.zeros((16,), jnp.float32)

    pltpu.sync_copy(src_hbm.at[pl.ds(rbase, NCH)], src_v)
    pltpu.sync_copy(dst_hbm.at[pl.ds(rbase, NCH)], dst_v)
    pltpu.sync_copy(s_hbm, snode_v)
    pltpu.sync_copy(d_hbm, acc_v)

    for j in range(NODE_SLICE // 16):
        zvec[pl.ds(j * 16, 16)] = zero16

    iota16 = lax.broadcasted_iota(jnp.int32, (16,), 0)

    def id_body(t, carry):
        for j in range(K // 16):
            idx_id[t, pl.ds(j * 16, 16)] = t * K + j * 16 + iota16
        return carry
    lax.fori_loop(0, NP // K, id_body, 0)

    def a_body(c2, carry):
        for j in range(8):
            sl = pl.ds(j * 16, 16)
            e = (plsc.load_gather(snode_v, [src_v[c2, sl]])
                 + plsc.load_gather(acc_v, [dst_v[c2, sl]]))
            e = jnp.where(e >= 0.0, e, e * 0.2)
            alpha_v[c2, sl] = jnp.exp(e)
        return carry
    lax.fori_loop(0, NCH, a_body, 0)

    for cc in range(EW // K, NCH):
        j0 = (EW - cc * K) // 16 if cc * K < EW else 0
        for j in range(j0, 8):
            alpha_v[cc, pl.ds(j * 16, 16)] = zero16

    def seg_pass(idx_v, acc_sh):
        pltpu.sync_copy(zvec, acc_sh.at[pl.ds(nbase, NODE_SLICE)])

        def lz_body(t, carry):
            lacc_v[pl.ds(t * 16, 16)] = zero16
            return carry
        lax.fori_loop(0, NP // 16, lz_body, 0)

        def lacc_body(c2, carry):
            for j in range(8):
                sl = pl.ds(j * 16, 16)
                plsc.addupdate_scatter(lacc_v, [idx_v[c2, sl]],
                                       alpha_v[c2, sl])
            return carry
        lax.fori_loop(0, NCH, lacc_body, 0)
        plsc.subcore_barrier()

        def scat_body(t, carry):
            pltpu.async_copy(lacc_v.at[pl.ds(t * K, K)],
                             acc_sh.at[idx_id.at[t]], ssem, add=True)
            return carry
        lax.fori_loop(0, NP // K, scat_body, 0)
        pltpu.make_async_copy(s_hbm, acc_v, ssem).wait()
        plsc.subcore_barrier()
        pltpu.sync_copy(acc_sh, acc_v)

        def rec_body(c2, carry):
            sl = pl.ds(c2 * 16, 16)
            acc_v[sl] = 1.0 / (acc_v[sl] + 1e-9)
            return carry
        lax.fori_loop(0, NP // 16, rec_body, 0)

        def div_body(c2, carry):
            for j in range(8):
                sl = pl.ds(j * 16, 16)
                r = plsc.load_gather(acc_v, [idx_v[c2, sl]])
                alpha_v[c2, sl] = alpha_v[c2, sl] * r
            return carry
        lax.fori_loop(0, NCH, div_body, 0)

    for _ in range(SINK_ITERS):
        seg_pass(dst_v, acc_shA)
        seg_pass(src_v, acc_shB)

    pltpu.sync_copy(alpha_v, alpha_hbm.at[pl.ds(rbase, NCH)])


_sc_attn = functools.partial(
    pl.kernel,
    out_type=jax.ShapeDtypeStruct((NROW, K), jnp.float32),
    mesh=plsc.VectorSubcoreMesh(
        core_axis_name="c", subcore_axis_name="s", num_cores=1),
    compiler_params=_SC_PARAMS,
    scratch_types=[
        pltpu.VMEM((NCH, K), jnp.int32),
        pltpu.VMEM((NCH, K), jnp.int32),
        pltpu.VMEM((NCH, K), jnp.float32),
        pltpu.VMEM((NP,), jnp.float32),
        pltpu.VMEM((NP,), jnp.float32),
        pltpu.VMEM((NP,), jnp.float32),
        pltpu.VMEM((NP // K, K), jnp.int32),
        pltpu.VMEM((NODE_SLICE,), jnp.float32),
        pltpu.VMEM_SHARED((NP,), jnp.float32),
        pltpu.VMEM_SHARED((NP,), jnp.float32),
        pltpu.SemaphoreType.DMA,
    ],
)(_sc_attn_body)


def _sc_agg_body(hlo_hbm, hhi_hbm, alpha_hbm, src_hbm, dst_hbm,
                 plo_hbm, phi_hbm,
                 src_v, dst_v, alpha_v, rowbuf, rowbuf2, zvec,
                 out_sh, h_sh, gsem0, gsem1):
    cid = lax.axis_index("c")
    sid = lax.axis_index("s")
    rbase = (cid * NSUB + sid) * NCH2
    nbase = sid * NODE_SLICE
    zero16 = jnp.zeros((16,), jnp.float32)

    pltpu.sync_copy(src_hbm.at[pl.ds(rbase, NCH2)], src_v)
    pltpu.sync_copy(dst_hbm.at[pl.ds(rbase, NCH2)], dst_v)
    pltpu.sync_copy(alpha_hbm.at[pl.ds(rbase, NCH2)], alpha_v)

    for h_hbm, p_hbm in ((hlo_hbm, plo_hbm), (hhi_hbm, phi_hbm)):
        pltpu.sync_copy(h_hbm.at[pl.ds(nbase, NODE_SLICE)],
                        h_sh.at[pl.ds(nbase, NODE_SLICE)])

        def z_body(i, carry):
            for j in range(HH // 16):
                rowbuf[i, pl.ds(j * 16, 16)] = zero16
            return carry
        lax.fori_loop(0, K, z_body, 0)
        for t in range(NODE_SLICE // K):
            pltpu.sync_copy(rowbuf, out_sh.at[pl.ds(nbase + t * K, K)])
        plsc.subcore_barrier()

        bufs = ((rowbuf, gsem0), (rowbuf2, gsem1))
        for b, (buf, gsem) in enumerate(bufs):
            pltpu.async_copy(h_sh.at[src_v.at[b]], buf, gsem)

        def pipe_body(g2, carry):
            for b, (buf, gsem) in enumerate(bufs):
                c2 = 2 * g2 + b
                pltpu.make_async_copy(h_sh.at[src_v.at[c2]], buf,
                                      gsem).wait()

                c2v = jnp.full((16,), c2, jnp.int32)

                def grp_body(g, carry2):
                    gbase = jnp.full((16,), g * 16, jnp.int32)
                    for l in range(16):
                        e2 = g * 16 + l
                        a16 = plsc.load_gather(alpha_v, [c2v, gbase + l])
                        for j in range(HH // 16):
                            sl = pl.ds(j * 16, 16)
                            buf[e2, sl] = buf[e2, sl] * a16
                    return carry2
                lax.fori_loop(0, K // 16, grp_body, 0)
                pltpu.sync_copy(buf, out_sh.at[dst_v.at[c2]], add=True)

                @pl.when(c2 + 2 < NCH2)
                def _():
                    pltpu.async_copy(h_sh.at[src_v.at[c2 + 2]], buf, gsem)
            return carry
        lax.fori_loop(0, NCH2 // 2, pipe_body, 0)
        plsc.subcore_barrier()

        for t in range(NODE_SLICE // K):
            pltpu.sync_copy(out_sh.at[pl.ds(nbase + t * K, K)],
                            p_hbm.at[cid, pl.ds(nbase + t * K, K)])
        plsc.subcore_barrier()


_sc_agg = functools.partial(
    pl.kernel,
    out_type=(
        jax.ShapeDtypeStruct((NCORE, NP, HH), jnp.float32),
        jax.ShapeDtypeStruct((NCORE, NP, HH), jnp.float32),
    ),
    mesh=plsc.VectorSubcoreMesh(
        core_axis_name="c", subcore_axis_name="s", num_cores=NCORE),
    compiler_params=_SC_PARAMS,
    scratch_types=[
        pltpu.VMEM((NCH2, K), jnp.int32),
        pltpu.VMEM((NCH2, K), jnp.int32),
        pltpu.VMEM((NCH2, K), jnp.float32),
        pltpu.VMEM((K, HH), jnp.float32),
        pltpu.VMEM((K, HH), jnp.float32),
        pltpu.VMEM((K,), jnp.float32),
        pltpu.VMEM_SHARED((NP, HH), jnp.float32),
        pltpu.VMEM_SHARED((NP, HH), jnp.float32),
        pltpu.SemaphoreType.DMA,
        pltpu.SemaphoreType.DMA,
    ],
)(_sc_agg_body)


def kernel(x, edge_index, batch_sample_indices,
           W0, a_src0, a_dst0, W1, a_src1, a_dst1, hw0, hb0, hw1, hb1):
    src = edge_index[0].astype(jnp.int32)
    dst = edge_index[1].astype(jnp.int32)
    src2d = jnp.pad(src.reshape(NSUB, EW), ((0, 0), (0, EWP - EW))
                    ).reshape(NROW, K)
    dst2d = jnp.pad(dst.reshape(NSUB, EW), ((0, 0), (0, EWP - EW))
                    ).reshape(NROW, K)
    xp = jnp.pad(x, ((0, NP - N), (0, 0)))
    A20 = jnp.pad(jnp.stack([a_src0, a_dst0], axis=1), ((0, 0), (0, 126)))
    A21 = jnp.pad(jnp.stack([a_src1, a_dst1], axis=1), ((0, 0), (0, 126)))

    h0lo, h0hi, sd0 = _tc_layer(xp[:, :HH], xp[:, HH:], W0, A20,
                                partials=False)
    alpha0 = _sc_attn(sd0[:, 0], sd0[:, 1], src2d, dst2d)
    p0lo, p0hi = _sc_agg(h0lo, h0hi, alpha0, src2d, dst2d)
    h1lo, h1hi, sd1 = _tc_layer(p0lo, p0hi, W1, A21, partials=True)
    alpha1 = _sc_attn(sd1[:, 0], sd1[:, 1], src2d, dst2d)
    p1lo, p1hi = _sc_agg(h1lo, h1hi, alpha1, src2d, dst2d)

    batch3d = jnp.pad(batch_sample_indices.astype(jnp.int32), (0, NP - N),
                      constant_values=G + 1).reshape(N_ROW_BLKS, 1, ROW_BLK)
    hb0r = hb0.reshape(1, 128)
    hw1p = jnp.pad(hw1, ((0, 0), (0, 128 - C)))
    hb1p = jnp.pad(hb1, (0, 128 - C)).reshape(1, 128)
    outp = _tc_tail(p1lo, p1hi, batch3d, hw0, hb0r, hw1p, hb1p)
    return outp[:, :C]
